# baseline (speedup 1.0000x reference)
import functools

import jax
import jax.numpy as jnp
from jax.experimental import pallas as pl
from jax.experimental.pallas import tpu as pltpu
from jax.experimental.pallas import tpu_sc as plsc

_N = 16384
_M = 4096
_K = 16
_R = 128
_C = 128
_QB = 128
_NW = 32
_SCQ = _M // _NW


def _fps_body(px_ref, py_ref, pz_ref, idx_ref, npx_ref, npy_ref, npz_ref):
    giota = (jax.lax.broadcasted_iota(jnp.int32, (_R, _C), 0) * _C
             + jax.lax.broadcasted_iota(jnp.int32, (_R, _C), 1))
    px = px_ref[...]
    py = py_ref[...]
    pz = pz_ref[...]

    oh0 = giota == 0
    neg_inf = jnp.float32(-jnp.inf)
    lx0 = jnp.max(jnp.where(oh0, px, neg_inf), axis=(0, 1), keepdims=True)
    ly0 = jnp.max(jnp.where(oh0, py, neg_inf), axis=(0, 1), keepdims=True)
    lz0 = jnp.max(jnp.where(oh0, pz, neg_inf), axis=(0, 1), keepdims=True)
    idx_ref[0:1, 0:1] = jnp.zeros((1, 1), jnp.int32)
    npx_ref[0:1, 0:1] = lx0
    npy_ref[0:1, 0:1] = ly0
    npz_ref[0:1, 0:1] = lz0

    s0 = jnp.full((_R, _C), jnp.inf, jnp.float32)

    def body(i, carry):
        s, lx, ly, lz = carry
        dx = px - lx
        dy = py - ly
        dz = pz - lz
        d = (dx * dx + dz * dz) + dy * dy
        s = jnp.minimum(s, d)
        maxv = jnp.max(s, axis=(0, 1), keepdims=True)
        cand = jnp.where(s == maxv, giota, jnp.int32(2 ** 30))
        nxtv = jnp.min(cand, axis=(0, 1), keepdims=True)
        oh = giota == nxtv
        nlx = jnp.max(jnp.where(oh, px, neg_inf), axis=(0, 1), keepdims=True)
        nly = jnp.max(jnp.where(oh, py, neg_inf), axis=(0, 1), keepdims=True)
        nlz = jnp.max(jnp.where(oh, pz, neg_inf), axis=(0, 1), keepdims=True)
        idx_ref[pl.ds(i, 1), :] = nxtv
        npx_ref[pl.ds(i, 1), :] = nlx
        npy_ref[pl.ds(i, 1), :] = nly
        npz_ref[pl.ds(i, 1), :] = nlz
        return s, nlx, nly, nlz

    jax.lax.fori_loop(1, _M, body, (s0, lx0, ly0, lz0), unroll=3)


def _run_fps(p):
    px = p[:, 0].reshape(_R, _C)
    py = p[:, 1].reshape(_R, _C)
    pz = p[:, 2].reshape(_R, _C)
    out_shapes = (
        jax.ShapeDtypeStruct((_M, 1), jnp.int32),
        jax.ShapeDtypeStruct((_M, 1), jnp.float32),
        jax.ShapeDtypeStruct((_M, 1), jnp.float32),
        jax.ShapeDtypeStruct((_M, 1), jnp.float32),
    )
    idx, npx, npy, npz = pl.pallas_call(
        _fps_body,
        out_shape=out_shapes,
    )(px, py, pz)
    n_p = jnp.concatenate([npx, npy, npz], axis=1)
    return idx[:, 0], n_p


def _knn_body(q_ref, pt_ref, nbr_ref):
    q = q_ref[...]
    pt = pt_ref[...]
    qq = (q[:, 0:1] * q[:, 0:1] + q[:, 2:3] * q[:, 2:3]) + q[:, 1:2] * q[:, 1:2]
    pp = (pt[0:1, :] * pt[0:1, :] + pt[2:3, :] * pt[2:3, :]) + pt[1:2, :] * pt[1:2, :]
    qp = jax.lax.dot_general(q, pt, (((1,), (0,)), ((), ())),
                             preferred_element_type=jnp.float32)
    d2 = (qq + pp) - 2.0 * qp
    giota = jax.lax.broadcasted_iota(jnp.int32, (_QB, _N), 1)
    big = jnp.int32(2 ** 30)
    for j in range(_K):
        minv = jnp.min(d2, axis=1, keepdims=True)
        cand = jnp.where(d2 == minv, giota, big)
        nxt = jnp.min(cand, axis=1, keepdims=True)
        nbr_ref[:, j:j + 1] = nxt
        d2 = jnp.where(giota == nxt, jnp.inf, d2)


def _run_knn(n_p, p):
    pt = p.T
    nbr = pl.pallas_call(
        _knn_body,
        grid=(_M // _QB,),
        in_specs=[
            pl.BlockSpec((_QB, 3), lambda i: (i, 0)),
            pl.BlockSpec((3, _N), lambda i: (0, 0)),
        ],
        out_specs=pl.BlockSpec((_QB, _K), lambda i: (i, 0)),
        out_shape=jax.ShapeDtypeStruct((_M, _K), jnp.int32),
    )(n_p, pt)
    return nbr


def _sc_gather_body(nbr_hbm, ntf_hbm, px_hbm, py_hbm, pz_hbm,
                    npx_hbm, npy_hbm, npz_hbm, x_hbm,
                    pooled_hbm, mpx_hbm, mpy_hbm, mpz_hbm, mx2_hbm,
                    nbr_v, ntf_v, gx_v, gy_v, gz_v,
                    npx_v, npy_v, npz_v,
                    rows_v, orow_v, mpx_v, mpy_v, mpz_v, mx2_v,
                    sem_g, sem_o, sem_c):
    wid = jax.lax.axis_index("s") * 2 + jax.lax.axis_index("c")
    base = wid * _SCQ

    pltpu.sync_copy(nbr_hbm.at[wid], nbr_v)
    pltpu.sync_copy(ntf_hbm.at[wid], ntf_v)
    pltpu.sync_copy(npx_hbm.at[pl.ds(base, _SCQ)], npx_v)
    pltpu.sync_copy(npy_hbm.at[pl.ds(base, _SCQ)], npy_v)
    pltpu.sync_copy(npz_hbm.at[pl.ds(base, _SCQ)], npz_v)

    for c in range(_K):
        pltpu.async_copy(px_hbm.at[ntf_v.at[c]], gx_v.at[c], sem_c)
        pltpu.async_copy(py_hbm.at[ntf_v.at[c]], gy_v.at[c], sem_c)
        pltpu.async_copy(pz_hbm.at[ntf_v.at[c]], gz_v.at[c], sem_c)
    for c in range(_K):
        pltpu.make_async_copy(px_hbm.at[ntf_v.at[c]], gx_v.at[c], sem_c).wait()
        pltpu.make_async_copy(py_hbm.at[ntf_v.at[c]], gy_v.at[c], sem_c).wait()
        pltpu.make_async_copy(pz_hbm.at[ntf_v.at[c]], gz_v.at[c], sem_c).wait()

    neg_inf = jnp.float32(-jnp.inf)
    for g in range(_SCQ // 16):
        qs = pl.ds(g * 16, 16)
        qx = npx_v[qs]
        qy = npy_v[qs]
        qz = npz_v[qs]
        mpx = jnp.full((16,), neg_inf, jnp.float32)
        mpy = jnp.full((16,), neg_inf, jnp.float32)
        mpz = jnp.full((16,), neg_inf, jnp.float32)
        mx2 = jnp.full((16,), neg_inf, jnp.float32)
        for j in range(_K):
            dx = gx_v[j, qs] - qx
            dy = gy_v[j, qs] - qy
            dz = gz_v[j, qs] - qz
            nrm2 = (dx * dx + dy * dy) + dz * dz
            mpx = jnp.maximum(mpx, dx)
            mpy = jnp.maximum(mpy, dy)
            mpz = jnp.maximum(mpz, dz)
            mx2 = jnp.maximum(mx2, nrm2)
        mpx_v[qs] = mpx
        mpy_v[qs] = mpy
        mpz_v[qs] = mpz
        mx2_v[qs] = mx2
    pltpu.sync_copy(mpx_v, mpx_hbm.at[pl.ds(base, _SCQ)])
    pltpu.sync_copy(mpy_v, mpy_hbm.at[pl.ds(base, _SCQ)])
    pltpu.sync_copy(mpz_v, mpz_hbm.at[pl.ds(base, _SCQ)])
    pltpu.sync_copy(mx2_v, mx2_hbm.at[pl.ds(base, _SCQ)])

    def issue(q, slot):
        pltpu.async_copy(x_hbm.at[nbr_v.at[q]], rows_v.at[slot], sem_g)

    issue(0, 0)

    def qloop(q, _):
        slot = jax.lax.rem(q, 2)
        pltpu.make_async_copy(x_hbm.at[nbr_v.at[q]], rows_v.at[slot],
                              sem_g).wait()

        @pl.when(q + 1 < _SCQ)
        def _():
            issue(q + 1, 1 - slot)

        @pl.when(q >= 2)
        def _():
            pltpu.make_async_copy(orow_v.at[slot],
                                  pooled_hbm.at[base + q - 2], sem_o).wait()

        def cloop(c, _):
            cs = pl.ds(c * 16, 16)
            acc = rows_v[slot, 0, cs]
            for r in range(1, _K):
                acc = jnp.maximum(acc, rows_v[slot, r, cs])
            orow_v[slot, cs] = acc
            return 0

        jax.lax.fori_loop(0, 512 // 16, cloop, 0)

        pltpu.async_copy(orow_v.at[slot], pooled_hbm.at[base + q], sem_o)
        return 0

    jax.lax.fori_loop(0, _SCQ, qloop, 0)
    pltpu.make_async_copy(orow_v.at[0], pooled_hbm.at[base + _SCQ - 2],
                          sem_o).wait()
    pltpu.make_async_copy(orow_v.at[1], pooled_hbm.at[base + _SCQ - 1],
                          sem_o).wait()


def _run_sc_gather(nbr, p, n_p, x):
    nbr_b = nbr.reshape(_NW, _SCQ, _K)
    ntf_b = jnp.transpose(nbr_b, (0, 2, 1))
    px = p[:, 0]
    py = p[:, 1]
    pz = p[:, 2]
    npx = n_p[:, 0]
    npy = n_p[:, 1]
    npz = n_p[:, 2]
    mesh = plsc.VectorSubcoreMesh(core_axis_name="c", subcore_axis_name="s")
    f32 = jnp.float32
    kern = pl.kernel(
        _sc_gather_body,
        out_type=[
            jax.ShapeDtypeStruct((_M, 512), f32),
            jax.ShapeDtypeStruct((_M,), f32),
            jax.ShapeDtypeStruct((_M,), f32),
            jax.ShapeDtypeStruct((_M,), f32),
            jax.ShapeDtypeStruct((_M,), f32),
        ],
        mesh=mesh,
        scratch_types=[
            pltpu.VMEM((_SCQ, _K), jnp.int32),
            pltpu.VMEM((_K, _SCQ), jnp.int32),
            pltpu.VMEM((_K, _SCQ), f32),
            pltpu.VMEM((_K, _SCQ), f32),
            pltpu.VMEM((_K, _SCQ), f32),
            pltpu.VMEM((_SCQ,), f32),
            pltpu.VMEM((_SCQ,), f32),
            pltpu.VMEM((_SCQ,), f32),
            pltpu.VMEM((2, _K, 512), f32),
            pltpu.VMEM((2, 512), f32),
            pltpu.VMEM((_SCQ,), f32),
            pltpu.VMEM((_SCQ,), f32),
            pltpu.VMEM((_SCQ,), f32),
            pltpu.VMEM((_SCQ,), f32),
            pltpu.SemaphoreType.DMA,
            pltpu.SemaphoreType.DMA,
            pltpu.SemaphoreType.DMA,
        ],
    )
    return kern(nbr_b, ntf_b, px, py, pz, npx, npy, npz, x)


def _mlp_body(pooled_ref, mpx_ref, mpy_ref, mpz_ref, mx2_ref,
              w3_ref, wx_ref, b_ref, gamma_ref, beta_ref, out_ref):
    mx2 = mx2_ref[...]
    denom = jnp.sqrt(mx2) + jnp.float32(1e-8)
    f3 = jnp.concatenate(
        [mpx_ref[...] / denom, mpy_ref[...] / denom, mpz_ref[...] / denom],
        axis=1)
    h3 = jax.lax.dot_general(f3, w3_ref[...], (((1,), (0,)), ((), ())),
                             preferred_element_type=jnp.float32)
    hx = jax.lax.dot_general(pooled_ref[...], wx_ref[...],
                             (((1,), (0,)), ((), ())),
                             preferred_element_type=jnp.float32)
    h = h3 + hx + b_ref[...]
    mean = jnp.mean(h, axis=0, keepdims=True)
    c = h - mean
    var = jnp.mean(c * c, axis=0, keepdims=True)
    hn = c / jnp.sqrt(var + 1e-5) * gamma_ref[...] + beta_ref[...]
    out_ref[...] = jnp.maximum(hn, 0.0)


def _run_mlp(pooled, mpx, mpy, mpz, mx2, W, b, gamma, beta):
    w3 = W[:3]
    wx = W[3:]
    out = pl.pallas_call(
        _mlp_body,
        out_shape=jax.ShapeDtypeStruct((_M, 512), jnp.float32),
    )(pooled, mpx.reshape(_M, 1), mpy.reshape(_M, 1), mpz.reshape(_M, 1),
      mx2.reshape(_M, 1), w3, wx, b.reshape(1, 512),
      gamma.reshape(1, 512), beta.reshape(1, 512))
    return out


def kernel(p, x, o, W, b, gamma, beta):
    idx, n_p = _run_fps(p)
    nbr = _run_knn(n_p, p)
    pooled, mpx, mpy, mpz, mx2 = _run_sc_gather(nbr, p, n_p, x)
    out = _run_mlp(pooled, mpx, mpy, mpz, mx2, W, b, gamma, beta)
    n_o = o // 4
    return (n_p, out, n_o)


# speedup vs baseline: 1.0149x; 1.0149x over previous
import functools

import jax
import jax.numpy as jnp
from jax.experimental import pallas as pl
from jax.experimental.pallas import tpu as pltpu
from jax.experimental.pallas import tpu_sc as plsc

_N = 16384
_M = 4096
_K = 16
_R = 128
_C = 128
_QB = 128
_NW = 32
_SCQ = _M // _NW


def _fps_body(px_ref, py_ref, pz_ref, idx_ref, npx_ref, npy_ref, npz_ref):
    giota = (jax.lax.broadcasted_iota(jnp.int32, (_R, _C), 0) * _C
             + jax.lax.broadcasted_iota(jnp.int32, (_R, _C), 1))
    px = px_ref[...]
    py = py_ref[...]
    pz = pz_ref[...]

    oh0 = giota == 0
    lx0 = jnp.sum(jnp.where(oh0, px, 0.0))
    ly0 = jnp.sum(jnp.where(oh0, py, 0.0))
    lz0 = jnp.sum(jnp.where(oh0, pz, 0.0))
    idx_ref[0:1, 0:1] = jnp.zeros((1, 1), jnp.int32)
    npx_ref[0:1, 0:1] = jnp.full((1, 1), lx0, jnp.float32)
    npy_ref[0:1, 0:1] = jnp.full((1, 1), ly0, jnp.float32)
    npz_ref[0:1, 0:1] = jnp.full((1, 1), lz0, jnp.float32)

    s0 = jnp.full((_R, _C), jnp.inf, jnp.float32)
    liota = jax.lax.broadcasted_iota(jnp.int32, (1, _C), 1)

    def body(i, carry):
        s, lx, ly, lz = carry
        dx = px - lx
        dy = py - ly
        dz = pz - lz
        d = (dx * dx + dz * dz) + dy * dy
        s = jnp.minimum(s, d)
        maxv = jnp.max(s)
        cand = jnp.where(s == maxv, giota, jnp.int32(2 ** 30))
        nxt = jnp.min(cand)
        row = jax.lax.shift_right_logical(nxt, 7)
        col = jax.lax.bitwise_and(nxt, jnp.int32(_C - 1))
        lane_oh = liota == col
        nlx = jnp.sum(jnp.where(lane_oh, px_ref[pl.ds(row, 1), :], 0.0))
        nly = jnp.sum(jnp.where(lane_oh, py_ref[pl.ds(row, 1), :], 0.0))
        nlz = jnp.sum(jnp.where(lane_oh, pz_ref[pl.ds(row, 1), :], 0.0))
        idx_ref[pl.ds(i, 1), :] = jnp.full((1, 1), nxt, jnp.int32)
        npx_ref[pl.ds(i, 1), :] = jnp.full((1, 1), nlx, jnp.float32)
        npy_ref[pl.ds(i, 1), :] = jnp.full((1, 1), nly, jnp.float32)
        npz_ref[pl.ds(i, 1), :] = jnp.full((1, 1), nlz, jnp.float32)
        return s, nlx, nly, nlz

    jax.lax.fori_loop(1, _M, body, (s0, lx0, ly0, lz0), unroll=3)


def _run_fps(p):
    px = p[:, 0].reshape(_R, _C)
    py = p[:, 1].reshape(_R, _C)
    pz = p[:, 2].reshape(_R, _C)
    out_shapes = (
        jax.ShapeDtypeStruct((_M, 1), jnp.int32),
        jax.ShapeDtypeStruct((_M, 1), jnp.float32),
        jax.ShapeDtypeStruct((_M, 1), jnp.float32),
        jax.ShapeDtypeStruct((_M, 1), jnp.float32),
    )
    idx, npx, npy, npz = pl.pallas_call(
        _fps_body,
        out_shape=out_shapes,
    )(px, py, pz)
    n_p = jnp.concatenate([npx, npy, npz], axis=1)
    return idx[:, 0], n_p


def _knn_body(q_ref, pt_ref, nbr_ref):
    q = q_ref[...]
    pt = pt_ref[...]
    qq = (q[:, 0:1] * q[:, 0:1] + q[:, 2:3] * q[:, 2:3]) + q[:, 1:2] * q[:, 1:2]
    pp = (pt[0:1, :] * pt[0:1, :] + pt[2:3, :] * pt[2:3, :]) + pt[1:2, :] * pt[1:2, :]
    qp = jax.lax.dot_general(q, pt, (((1,), (0,)), ((), ())),
                             preferred_element_type=jnp.float32)
    d2 = (qq + pp) - 2.0 * qp
    giota = jax.lax.broadcasted_iota(jnp.int32, (_QB, _N), 1)
    big = jnp.int32(2 ** 30)
    nxt = jnp.full((_QB, 1), jnp.int32(-1))
    for j in range(_K):
        d2 = jnp.where(giota == nxt, jnp.inf, d2)
        minv = jnp.min(d2, axis=1, keepdims=True)
        cand = jnp.where(d2 == minv, giota, big)
        nxt = jnp.min(cand, axis=1, keepdims=True)
        nbr_ref[:, j:j + 1] = nxt


def _run_knn(n_p, p):
    pt = p.T
    nbr = pl.pallas_call(
        _knn_body,
        grid=(_M // _QB,),
        in_specs=[
            pl.BlockSpec((_QB, 3), lambda i: (i, 0)),
            pl.BlockSpec((3, _N), lambda i: (0, 0)),
        ],
        out_specs=pl.BlockSpec((_QB, _K), lambda i: (i, 0)),
        out_shape=jax.ShapeDtypeStruct((_M, _K), jnp.int32),
    )(n_p, pt)
    return nbr


def _sc_gather_body(nbr_hbm, ntf_hbm, px_hbm, py_hbm, pz_hbm,
                    npx_hbm, npy_hbm, npz_hbm, x_hbm,
                    pooled_hbm, mpx_hbm, mpy_hbm, mpz_hbm, mx2_hbm,
                    nbr_v, ntf_v, gx_v, gy_v, gz_v,
                    npx_v, npy_v, npz_v,
                    rows_v, orow_v, mpx_v, mpy_v, mpz_v, mx2_v,
                    sem_g, sem_o, sem_c):
    wid = jax.lax.axis_index("s") * 2 + jax.lax.axis_index("c")
    base = wid * _SCQ

    pltpu.sync_copy(nbr_hbm.at[wid], nbr_v)
    pltpu.sync_copy(ntf_hbm.at[wid], ntf_v)
    pltpu.sync_copy(npx_hbm.at[pl.ds(base, _SCQ)], npx_v)
    pltpu.sync_copy(npy_hbm.at[pl.ds(base, _SCQ)], npy_v)
    pltpu.sync_copy(npz_hbm.at[pl.ds(base, _SCQ)], npz_v)

    for c in range(_K):
        pltpu.async_copy(px_hbm.at[ntf_v.at[c]], gx_v.at[c], sem_c)
        pltpu.async_copy(py_hbm.at[ntf_v.at[c]], gy_v.at[c], sem_c)
        pltpu.async_copy(pz_hbm.at[ntf_v.at[c]], gz_v.at[c], sem_c)
    for c in range(_K):
        pltpu.make_async_copy(px_hbm.at[ntf_v.at[c]], gx_v.at[c], sem_c).wait()
        pltpu.make_async_copy(py_hbm.at[ntf_v.at[c]], gy_v.at[c], sem_c).wait()
        pltpu.make_async_copy(pz_hbm.at[ntf_v.at[c]], gz_v.at[c], sem_c).wait()

    neg_inf = jnp.float32(-jnp.inf)
    for g in range(_SCQ // 16):
        qs = pl.ds(g * 16, 16)
        qx = npx_v[qs]
        qy = npy_v[qs]
        qz = npz_v[qs]
        mpx = jnp.full((16,), neg_inf, jnp.float32)
        mpy = jnp.full((16,), neg_inf, jnp.float32)
        mpz = jnp.full((16,), neg_inf, jnp.float32)
        mx2 = jnp.full((16,), neg_inf, jnp.float32)
        for j in range(_K):
            dx = gx_v[j, qs] - qx
            dy = gy_v[j, qs] - qy
            dz = gz_v[j, qs] - qz
            nrm2 = (dx * dx + dy * dy) + dz * dz
            mpx = jnp.maximum(mpx, dx)
            mpy = jnp.maximum(mpy, dy)
            mpz = jnp.maximum(mpz, dz)
            mx2 = jnp.maximum(mx2, nrm2)
        mpx_v[qs] = mpx
        mpy_v[qs] = mpy
        mpz_v[qs] = mpz
        mx2_v[qs] = mx2
    pltpu.sync_copy(mpx_v, mpx_hbm.at[pl.ds(base, _SCQ)])
    pltpu.sync_copy(mpy_v, mpy_hbm.at[pl.ds(base, _SCQ)])
    pltpu.sync_copy(mpz_v, mpz_hbm.at[pl.ds(base, _SCQ)])
    pltpu.sync_copy(mx2_v, mx2_hbm.at[pl.ds(base, _SCQ)])

    def issue(q, slot):
        pltpu.async_copy(x_hbm.at[nbr_v.at[q]], rows_v.at[slot], sem_g)

    issue(0, 0)

    def qloop(q, _):
        slot = jax.lax.rem(q, 2)
        pltpu.make_async_copy(x_hbm.at[nbr_v.at[q]], rows_v.at[slot],
                              sem_g).wait()

        @pl.when(q + 1 < _SCQ)
        def _():
            issue(q + 1, 1 - slot)

        @pl.when(q >= 2)
        def _():
            pltpu.make_async_copy(orow_v.at[slot],
                                  pooled_hbm.at[base + q - 2], sem_o).wait()

        def cloop(c, _):
            cs = pl.ds(c * 16, 16)
            acc = rows_v[slot, 0, cs]
            for r in range(1, _K):
                acc = jnp.maximum(acc, rows_v[slot, r, cs])
            orow_v[slot, cs] = acc
            return 0

        jax.lax.fori_loop(0, 512 // 16, cloop, 0)

        pltpu.async_copy(orow_v.at[slot], pooled_hbm.at[base + q], sem_o)
        return 0

    jax.lax.fori_loop(0, _SCQ, qloop, 0)
    pltpu.make_async_copy(orow_v.at[0], pooled_hbm.at[base + _SCQ - 2],
                          sem_o).wait()
    pltpu.make_async_copy(orow_v.at[1], pooled_hbm.at[base + _SCQ - 1],
                          sem_o).wait()


def _run_sc_gather(nbr, p, n_p, x):
    nbr_b = nbr.reshape(_NW, _SCQ, _K)
    ntf_b = jnp.transpose(nbr_b, (0, 2, 1))
    px = p[:, 0]
    py = p[:, 1]
    pz = p[:, 2]
    npx = n_p[:, 0]
    npy = n_p[:, 1]
    npz = n_p[:, 2]
    mesh = plsc.VectorSubcoreMesh(core_axis_name="c", subcore_axis_name="s")
    f32 = jnp.float32
    kern = pl.kernel(
        _sc_gather_body,
        out_type=[
            jax.ShapeDtypeStruct((_M, 512), f32),
            jax.ShapeDtypeStruct((_M,), f32),
            jax.ShapeDtypeStruct((_M,), f32),
            jax.ShapeDtypeStruct((_M,), f32),
            jax.ShapeDtypeStruct((_M,), f32),
        ],
        mesh=mesh,
        scratch_types=[
            pltpu.VMEM((_SCQ, _K), jnp.int32),
            pltpu.VMEM((_K, _SCQ), jnp.int32),
            pltpu.VMEM((_K, _SCQ), f32),
            pltpu.VMEM((_K, _SCQ), f32),
            pltpu.VMEM((_K, _SCQ), f32),
            pltpu.VMEM((_SCQ,), f32),
            pltpu.VMEM((_SCQ,), f32),
            pltpu.VMEM((_SCQ,), f32),
            pltpu.VMEM((2, _K, 512), f32),
            pltpu.VMEM((2, 512), f32),
            pltpu.VMEM((_SCQ,), f32),
            pltpu.VMEM((_SCQ,), f32),
            pltpu.VMEM((_SCQ,), f32),
            pltpu.VMEM((_SCQ,), f32),
            pltpu.SemaphoreType.DMA,
            pltpu.SemaphoreType.DMA,
            pltpu.SemaphoreType.DMA,
        ],
    )
    return kern(nbr_b, ntf_b, px, py, pz, npx, npy, npz, x)


def _mlp_body(pooled_ref, mpx_ref, mpy_ref, mpz_ref, mx2_ref,
              w3_ref, wx_ref, b_ref, gamma_ref, beta_ref, out_ref):
    mx2 = mx2_ref[...]
    denom = jnp.sqrt(mx2) + jnp.float32(1e-8)
    f3 = jnp.concatenate(
        [mpx_ref[...] / denom, mpy_ref[...] / denom, mpz_ref[...] / denom],
        axis=1)
    h3 = jax.lax.dot_general(f3, w3_ref[...], (((1,), (0,)), ((), ())),
                             preferred_element_type=jnp.float32)
    hx = jax.lax.dot_general(pooled_ref[...], wx_ref[...],
                             (((1,), (0,)), ((), ())),
                             preferred_element_type=jnp.float32)
    h = h3 + hx + b_ref[...]
    mean = jnp.mean(h, axis=0, keepdims=True)
    c = h - mean
    var = jnp.mean(c * c, axis=0, keepdims=True)
    hn = c / jnp.sqrt(var + 1e-5) * gamma_ref[...] + beta_ref[...]
    out_ref[...] = jnp.maximum(hn, 0.0)


def _run_mlp(pooled, mpx, mpy, mpz, mx2, W, b, gamma, beta):
    w3 = W[:3]
    wx = W[3:]
    out = pl.pallas_call(
        _mlp_body,
        out_shape=jax.ShapeDtypeStruct((_M, 512), jnp.float32),
    )(pooled, mpx.reshape(_M, 1), mpy.reshape(_M, 1), mpz.reshape(_M, 1),
      mx2.reshape(_M, 1), w3, wx, b.reshape(1, 512),
      gamma.reshape(1, 512), beta.reshape(1, 512))
    return out


def kernel(p, x, o, W, b, gamma, beta):
    idx, n_p = _run_fps(p)
    nbr = _run_knn(n_p, p)
    pooled, mpx, mpy, mpz, mx2 = _run_sc_gather(nbr, p, n_p, x)
    out = _run_mlp(pooled, mpx, mpy, mpz, mx2, W, b, gamma, beta)
    n_o = o // 4
    return (n_p, out, n_o)


# speedup vs baseline: 1.0154x; 1.0005x over previous
import functools

import jax
import jax.numpy as jnp
from jax.experimental import pallas as pl
from jax.experimental.pallas import tpu as pltpu
from jax.experimental.pallas import tpu_sc as plsc

_N = 16384
_M = 4096
_K = 16
_R = 128
_C = 128
_QB = 128
_NW = 32
_SCQ = _M // _NW


def _fps_body(px_ref, py_ref, pz_ref, idx_ref):
    giota = (jax.lax.broadcasted_iota(jnp.int32, (_R, _C), 0) * _C
             + jax.lax.broadcasted_iota(jnp.int32, (_R, _C), 1))
    px = px_ref[...]
    py = py_ref[...]
    pz = pz_ref[...]

    oh0 = giota == 0
    lx0 = jnp.sum(jnp.where(oh0, px, 0.0))
    ly0 = jnp.sum(jnp.where(oh0, py, 0.0))
    lz0 = jnp.sum(jnp.where(oh0, pz, 0.0))
    idx_ref[0:1, 0:1] = jnp.zeros((1, 1), jnp.int32)

    s0 = jnp.full((_R, _C), jnp.inf, jnp.float32)
    liota = jax.lax.broadcasted_iota(jnp.int32, (1, _C), 1)

    def body(i, carry):
        s, lx, ly, lz = carry
        dx = px - lx
        dy = py - ly
        dz = pz - lz
        d = (dx * dx + dz * dz) + dy * dy
        s = jnp.minimum(s, d)
        maxv = jnp.max(s)
        cand = jnp.where(s == maxv, giota, jnp.int32(2 ** 30))
        nxt = jnp.min(cand)
        row = jax.lax.shift_right_logical(nxt, 7)
        col = jax.lax.bitwise_and(nxt, jnp.int32(_C - 1))
        lane_oh = liota == col
        nlx = jnp.sum(jnp.where(lane_oh, px_ref[pl.ds(row, 1), :], 0.0))
        nly = jnp.sum(jnp.where(lane_oh, py_ref[pl.ds(row, 1), :], 0.0))
        nlz = jnp.sum(jnp.where(lane_oh, pz_ref[pl.ds(row, 1), :], 0.0))
        idx_ref[pl.ds(i, 1), :] = jnp.full((1, 1), nxt, jnp.int32)
        return s, nlx, nly, nlz

    jax.lax.fori_loop(1, _M, body, (s0, lx0, ly0, lz0), unroll=3)


def _run_fps(p):
    px = p[:, 0].reshape(_R, _C)
    py = p[:, 1].reshape(_R, _C)
    pz = p[:, 2].reshape(_R, _C)
    idx = pl.pallas_call(
        _fps_body,
        out_shape=jax.ShapeDtypeStruct((_M, 1), jnp.int32),
    )(px, py, pz)
    idx = idx[:, 0]
    n_p = p[idx]
    return idx, n_p


def _knn_body(q_ref, pt_ref, nbr_ref):
    q = q_ref[...]
    pt = pt_ref[...]
    qq = (q[:, 0:1] * q[:, 0:1] + q[:, 2:3] * q[:, 2:3]) + q[:, 1:2] * q[:, 1:2]
    pp = (pt[0:1, :] * pt[0:1, :] + pt[2:3, :] * pt[2:3, :]) + pt[1:2, :] * pt[1:2, :]
    qp = jax.lax.dot_general(q, pt, (((1,), (0,)), ((), ())),
                             preferred_element_type=jnp.float32)
    d2 = (qq + pp) - 2.0 * qp
    giota = jax.lax.broadcasted_iota(jnp.int32, (_QB, _N), 1)
    big = jnp.int32(2 ** 30)
    nxt = jnp.full((_QB, 1), jnp.int32(-1))
    for j in range(_K):
        d2 = jnp.where(giota == nxt, jnp.inf, d2)
        minv = jnp.min(d2, axis=1, keepdims=True)
        cand = jnp.where(d2 == minv, giota, big)
        nxt = jnp.min(cand, axis=1, keepdims=True)
        nbr_ref[:, j:j + 1] = nxt


def _run_knn(n_p, p):
    pt = p.T
    nbr = pl.pallas_call(
        _knn_body,
        grid=(_M // _QB,),
        in_specs=[
            pl.BlockSpec((_QB, 3), lambda i: (i, 0)),
            pl.BlockSpec((3, _N), lambda i: (0, 0)),
        ],
        out_specs=pl.BlockSpec((_QB, _K), lambda i: (i, 0)),
        out_shape=jax.ShapeDtypeStruct((_M, _K), jnp.int32),
    )(n_p, pt)
    return nbr


def _sc_gather_body(nbr_hbm, ntf_hbm, px_hbm, py_hbm, pz_hbm,
                    npx_hbm, npy_hbm, npz_hbm, x_hbm,
                    pooled_hbm, mpx_hbm, mpy_hbm, mpz_hbm, mx2_hbm,
                    nbr_v, ntf_v, gx_v, gy_v, gz_v,
                    npx_v, npy_v, npz_v,
                    rows_v, orow_v, mpx_v, mpy_v, mpz_v, mx2_v,
                    sem_g, sem_o, sem_c):
    wid = jax.lax.axis_index("s") * 2 + jax.lax.axis_index("c")
    base = wid * _SCQ

    pltpu.sync_copy(nbr_hbm.at[wid], nbr_v)
    pltpu.sync_copy(ntf_hbm.at[wid], ntf_v)
    pltpu.sync_copy(npx_hbm.at[pl.ds(base, _SCQ)], npx_v)
    pltpu.sync_copy(npy_hbm.at[pl.ds(base, _SCQ)], npy_v)
    pltpu.sync_copy(npz_hbm.at[pl.ds(base, _SCQ)], npz_v)

    for c in range(_K):
        pltpu.async_copy(px_hbm.at[ntf_v.at[c]], gx_v.at[c], sem_c)
        pltpu.async_copy(py_hbm.at[ntf_v.at[c]], gy_v.at[c], sem_c)
        pltpu.async_copy(pz_hbm.at[ntf_v.at[c]], gz_v.at[c], sem_c)
    for c in range(_K):
        pltpu.make_async_copy(px_hbm.at[ntf_v.at[c]], gx_v.at[c], sem_c).wait()
        pltpu.make_async_copy(py_hbm.at[ntf_v.at[c]], gy_v.at[c], sem_c).wait()
        pltpu.make_async_copy(pz_hbm.at[ntf_v.at[c]], gz_v.at[c], sem_c).wait()

    neg_inf = jnp.float32(-jnp.inf)
    for g in range(_SCQ // 16):
        qs = pl.ds(g * 16, 16)
        qx = npx_v[qs]
        qy = npy_v[qs]
        qz = npz_v[qs]
        mpx = jnp.full((16,), neg_inf, jnp.float32)
        mpy = jnp.full((16,), neg_inf, jnp.float32)
        mpz = jnp.full((16,), neg_inf, jnp.float32)
        mx2 = jnp.full((16,), neg_inf, jnp.float32)
        for j in range(_K):
            dx = gx_v[j, qs] - qx
            dy = gy_v[j, qs] - qy
            dz = gz_v[j, qs] - qz
            nrm2 = (dx * dx + dy * dy) + dz * dz
            mpx = jnp.maximum(mpx, dx)
            mpy = jnp.maximum(mpy, dy)
            mpz = jnp.maximum(mpz, dz)
            mx2 = jnp.maximum(mx2, nrm2)
        mpx_v[qs] = mpx
        mpy_v[qs] = mpy
        mpz_v[qs] = mpz
        mx2_v[qs] = mx2
    pltpu.sync_copy(mpx_v, mpx_hbm.at[pl.ds(base, _SCQ)])
    pltpu.sync_copy(mpy_v, mpy_hbm.at[pl.ds(base, _SCQ)])
    pltpu.sync_copy(mpz_v, mpz_hbm.at[pl.ds(base, _SCQ)])
    pltpu.sync_copy(mx2_v, mx2_hbm.at[pl.ds(base, _SCQ)])

    def issue(q, slot):
        pltpu.async_copy(x_hbm.at[nbr_v.at[q]], rows_v.at[slot], sem_g)

    issue(0, 0)

    def qloop(q, _):
        slot = jax.lax.rem(q, 2)
        pltpu.make_async_copy(x_hbm.at[nbr_v.at[q]], rows_v.at[slot],
                              sem_g).wait()

        @pl.when(q + 1 < _SCQ)
        def _():
            issue(q + 1, 1 - slot)

        @pl.when(q >= 2)
        def _():
            pltpu.make_async_copy(orow_v.at[slot],
                                  pooled_hbm.at[base + q - 2], sem_o).wait()

        def cloop(c, _):
            cs = pl.ds(c * 16, 16)
            acc = rows_v[slot, 0, cs]
            for r in range(1, _K):
                acc = jnp.maximum(acc, rows_v[slot, r, cs])
            orow_v[slot, cs] = acc
            return 0

        jax.lax.fori_loop(0, 512 // 16, cloop, 0)

        pltpu.async_copy(orow_v.at[slot], pooled_hbm.at[base + q], sem_o)
        return 0

    jax.lax.fori_loop(0, _SCQ, qloop, 0)
    pltpu.make_async_copy(orow_v.at[0], pooled_hbm.at[base + _SCQ - 2],
                          sem_o).wait()
    pltpu.make_async_copy(orow_v.at[1], pooled_hbm.at[base + _SCQ - 1],
                          sem_o).wait()


def _run_sc_gather(nbr, p, n_p, x):
    nbr_b = nbr.reshape(_NW, _SCQ, _K)
    ntf_b = jnp.transpose(nbr_b, (0, 2, 1))
    px = p[:, 0]
    py = p[:, 1]
    pz = p[:, 2]
    npx = n_p[:, 0]
    npy = n_p[:, 1]
    npz = n_p[:, 2]
    mesh = plsc.VectorSubcoreMesh(core_axis_name="c", subcore_axis_name="s")
    f32 = jnp.float32
    kern = pl.kernel(
        _sc_gather_body,
        out_type=[
            jax.ShapeDtypeStruct((_M, 512), f32),
            jax.ShapeDtypeStruct((_M,), f32),
            jax.ShapeDtypeStruct((_M,), f32),
            jax.ShapeDtypeStruct((_M,), f32),
            jax.ShapeDtypeStruct((_M,), f32),
        ],
        mesh=mesh,
        scratch_types=[
            pltpu.VMEM((_SCQ, _K), jnp.int32),
            pltpu.VMEM((_K, _SCQ), jnp.int32),
            pltpu.VMEM((_K, _SCQ), f32),
            pltpu.VMEM((_K, _SCQ), f32),
            pltpu.VMEM((_K, _SCQ), f32),
            pltpu.VMEM((_SCQ,), f32),
            pltpu.VMEM((_SCQ,), f32),
            pltpu.VMEM((_SCQ,), f32),
            pltpu.VMEM((2, _K, 512), f32),
            pltpu.VMEM((2, 512), f32),
            pltpu.VMEM((_SCQ,), f32),
            pltpu.VMEM((_SCQ,), f32),
            pltpu.VMEM((_SCQ,), f32),
            pltpu.VMEM((_SCQ,), f32),
            pltpu.SemaphoreType.DMA,
            pltpu.SemaphoreType.DMA,
            pltpu.SemaphoreType.DMA,
        ],
    )
    return kern(nbr_b, ntf_b, px, py, pz, npx, npy, npz, x)


def _mlp_body(pooled_ref, mpx_ref, mpy_ref, mpz_ref, mx2_ref,
              w3_ref, wx_ref, b_ref, gamma_ref, beta_ref, out_ref):
    mx2 = mx2_ref[...]
    denom = jnp.sqrt(mx2) + jnp.float32(1e-8)
    f3 = jnp.concatenate(
        [mpx_ref[...] / denom, mpy_ref[...] / denom, mpz_ref[...] / denom],
        axis=1)
    h3 = jax.lax.dot_general(f3, w3_ref[...], (((1,), (0,)), ((), ())),
                             preferred_element_type=jnp.float32)
    hx = jax.lax.dot_general(pooled_ref[...], wx_ref[...],
                             (((1,), (0,)), ((), ())),
                             preferred_element_type=jnp.float32)
    h = h3 + hx + b_ref[...]
    mean = jnp.mean(h, axis=0, keepdims=True)
    c = h - mean
    var = jnp.mean(c * c, axis=0, keepdims=True)
    hn = c / jnp.sqrt(var + 1e-5) * gamma_ref[...] + beta_ref[...]
    out_ref[...] = jnp.maximum(hn, 0.0)


def _run_mlp(pooled, mpx, mpy, mpz, mx2, W, b, gamma, beta):
    w3 = W[:3]
    wx = W[3:]
    out = pl.pallas_call(
        _mlp_body,
        out_shape=jax.ShapeDtypeStruct((_M, 512), jnp.float32),
    )(pooled, mpx.reshape(_M, 1), mpy.reshape(_M, 1), mpz.reshape(_M, 1),
      mx2.reshape(_M, 1), w3, wx, b.reshape(1, 512),
      gamma.reshape(1, 512), beta.reshape(1, 512))
    return out


def kernel(p, x, o, W, b, gamma, beta):
    idx, n_p = _run_fps(p)
    nbr = _run_knn(n_p, p)
    pooled, mpx, mpy, mpz, mx2 = _run_sc_gather(nbr, p, n_p, x)
    out = _run_mlp(pooled, mpx, mpy, mpz, mx2, W, b, gamma, beta)
    n_o = o // 4
    return (n_p, out, n_o)


# speedup vs baseline: 1.0325x; 1.0168x over previous
import functools

import jax
import jax.numpy as jnp
from jax.experimental import pallas as pl
from jax.experimental.pallas import tpu as pltpu
from jax.experimental.pallas import tpu_sc as plsc

_N = 16384
_M = 4096
_K = 16
_R = 128
_C = 128
_QB = 128
_NW = 32
_SCQ = _M // _NW


def _fps_body(px_ref, py_ref, pz_ref, idx_ref):
    giota = (jax.lax.broadcasted_iota(jnp.int32, (_R, _C), 0) * _C
             + jax.lax.broadcasted_iota(jnp.int32, (_R, _C), 1))
    px = px_ref[...]
    py = py_ref[...]
    pz = pz_ref[...]

    oh0 = giota == 0
    lx0 = jnp.sum(jnp.where(oh0, px, 0.0))
    ly0 = jnp.sum(jnp.where(oh0, py, 0.0))
    lz0 = jnp.sum(jnp.where(oh0, pz, 0.0))
    idx_ref[0:1, 0:1] = jnp.zeros((1, 1), jnp.int32)

    s0 = jnp.full((_R, _C), jnp.inf, jnp.float32)
    liota = jax.lax.broadcasted_iota(jnp.int32, (1, _C), 1)

    def body(i, carry):
        s, lx, ly, lz = carry
        dx = px - lx
        dy = py - ly
        dz = pz - lz
        d = (dx * dx + dz * dz) + dy * dy
        s = jnp.minimum(s, d)
        nxt = jnp.argmax(s).astype(jnp.int32)
        row = jax.lax.shift_right_logical(nxt, 7)
        col = jax.lax.bitwise_and(nxt, jnp.int32(_C - 1))
        lane_oh = liota == col
        nlx = jnp.sum(jnp.where(lane_oh, px_ref[pl.ds(row, 1), :], 0.0))
        nly = jnp.sum(jnp.where(lane_oh, py_ref[pl.ds(row, 1), :], 0.0))
        nlz = jnp.sum(jnp.where(lane_oh, pz_ref[pl.ds(row, 1), :], 0.0))
        idx_ref[pl.ds(i, 1), :] = jnp.full((1, 1), nxt, jnp.int32)
        return s, nlx, nly, nlz

    jax.lax.fori_loop(1, _M, body, (s0, lx0, ly0, lz0), unroll=3)


def _run_fps(p):
    px = p[:, 0].reshape(_R, _C)
    py = p[:, 1].reshape(_R, _C)
    pz = p[:, 2].reshape(_R, _C)
    idx = pl.pallas_call(
        _fps_body,
        out_shape=jax.ShapeDtypeStruct((_M, 1), jnp.int32),
    )(px, py, pz)
    idx = idx[:, 0]
    n_p = p[idx]
    return idx, n_p


def _knn_body(q_ref, pt_ref, nbr_ref):
    q = q_ref[...]
    pt = pt_ref[...]
    qq = (q[:, 0:1] * q[:, 0:1] + q[:, 2:3] * q[:, 2:3]) + q[:, 1:2] * q[:, 1:2]
    pp = (pt[0:1, :] * pt[0:1, :] + pt[2:3, :] * pt[2:3, :]) + pt[1:2, :] * pt[1:2, :]
    qp = jax.lax.dot_general(q, pt, (((1,), (0,)), ((), ())),
                             preferred_element_type=jnp.float32)
    d2 = (qq + pp) - 2.0 * qp
    giota = jax.lax.broadcasted_iota(jnp.int32, (_QB, _N), 1)
    big = jnp.int32(2 ** 30)
    nxt = jnp.full((_QB, 1), jnp.int32(-1))
    for j in range(_K):
        d2 = jnp.where(giota == nxt, jnp.inf, d2)
        minv = jnp.min(d2, axis=1, keepdims=True)
        cand = jnp.where(d2 == minv, giota, big)
        nxt = jnp.min(cand, axis=1, keepdims=True)
        nbr_ref[:, j:j + 1] = nxt


def _run_knn(n_p, p):
    pt = p.T
    nbr = pl.pallas_call(
        _knn_body,
        grid=(_M // _QB,),
        in_specs=[
            pl.BlockSpec((_QB, 3), lambda i: (i, 0)),
            pl.BlockSpec((3, _N), lambda i: (0, 0)),
        ],
        out_specs=pl.BlockSpec((_QB, _K), lambda i: (i, 0)),
        out_shape=jax.ShapeDtypeStruct((_M, _K), jnp.int32),
    )(n_p, pt)
    return nbr


def _sc_gather_body(nbr_hbm, ntf_hbm, px_hbm, py_hbm, pz_hbm,
                    npx_hbm, npy_hbm, npz_hbm, x_hbm,
                    pooled_hbm, mpx_hbm, mpy_hbm, mpz_hbm, mx2_hbm,
                    nbr_v, ntf_v, gx_v, gy_v, gz_v,
                    npx_v, npy_v, npz_v,
                    rows_v, orow_v, mpx_v, mpy_v, mpz_v, mx2_v,
                    sem_g, sem_o, sem_c):
    wid = jax.lax.axis_index("s") * 2 + jax.lax.axis_index("c")
    base = wid * _SCQ

    pltpu.sync_copy(nbr_hbm.at[wid], nbr_v)
    pltpu.sync_copy(ntf_hbm.at[wid], ntf_v)
    pltpu.sync_copy(npx_hbm.at[pl.ds(base, _SCQ)], npx_v)
    pltpu.sync_copy(npy_hbm.at[pl.ds(base, _SCQ)], npy_v)
    pltpu.sync_copy(npz_hbm.at[pl.ds(base, _SCQ)], npz_v)

    for c in range(_K):
        pltpu.async_copy(px_hbm.at[ntf_v.at[c]], gx_v.at[c], sem_c)
        pltpu.async_copy(py_hbm.at[ntf_v.at[c]], gy_v.at[c], sem_c)
        pltpu.async_copy(pz_hbm.at[ntf_v.at[c]], gz_v.at[c], sem_c)
    for c in range(_K):
        pltpu.make_async_copy(px_hbm.at[ntf_v.at[c]], gx_v.at[c], sem_c).wait()
        pltpu.make_async_copy(py_hbm.at[ntf_v.at[c]], gy_v.at[c], sem_c).wait()
        pltpu.make_async_copy(pz_hbm.at[ntf_v.at[c]], gz_v.at[c], sem_c).wait()

    neg_inf = jnp.float32(-jnp.inf)
    for g in range(_SCQ // 16):
        qs = pl.ds(g * 16, 16)
        qx = npx_v[qs]
        qy = npy_v[qs]
        qz = npz_v[qs]
        mpx = jnp.full((16,), neg_inf, jnp.float32)
        mpy = jnp.full((16,), neg_inf, jnp.float32)
        mpz = jnp.full((16,), neg_inf, jnp.float32)
        mx2 = jnp.full((16,), neg_inf, jnp.float32)
        for j in range(_K):
            dx = gx_v[j, qs] - qx
            dy = gy_v[j, qs] - qy
            dz = gz_v[j, qs] - qz
            nrm2 = (dx * dx + dy * dy) + dz * dz
            mpx = jnp.maximum(mpx, dx)
            mpy = jnp.maximum(mpy, dy)
            mpz = jnp.maximum(mpz, dz)
            mx2 = jnp.maximum(mx2, nrm2)
        mpx_v[qs] = mpx
        mpy_v[qs] = mpy
        mpz_v[qs] = mpz
        mx2_v[qs] = mx2
    pltpu.sync_copy(mpx_v, mpx_hbm.at[pl.ds(base, _SCQ)])
    pltpu.sync_copy(mpy_v, mpy_hbm.at[pl.ds(base, _SCQ)])
    pltpu.sync_copy(mpz_v, mpz_hbm.at[pl.ds(base, _SCQ)])
    pltpu.sync_copy(mx2_v, mx2_hbm.at[pl.ds(base, _SCQ)])

    def issue(q, slot):
        pltpu.async_copy(x_hbm.at[nbr_v.at[q]], rows_v.at[slot], sem_g)

    issue(0, 0)

    def qloop(q, _):
        slot = jax.lax.rem(q, 2)
        pltpu.make_async_copy(x_hbm.at[nbr_v.at[q]], rows_v.at[slot],
                              sem_g).wait()

        @pl.when(q + 1 < _SCQ)
        def _():
            issue(q + 1, 1 - slot)

        @pl.when(q >= 2)
        def _():
            pltpu.make_async_copy(orow_v.at[slot],
                                  pooled_hbm.at[base + q - 2], sem_o).wait()

        def cloop(c, _):
            cs = pl.ds(c * 16, 16)
            acc = rows_v[slot, 0, cs]
            for r in range(1, _K):
                acc = jnp.maximum(acc, rows_v[slot, r, cs])
            orow_v[slot, cs] = acc
            return 0

        jax.lax.fori_loop(0, 512 // 16, cloop, 0)

        pltpu.async_copy(orow_v.at[slot], pooled_hbm.at[base + q], sem_o)
        return 0

    jax.lax.fori_loop(0, _SCQ, qloop, 0)
    pltpu.make_async_copy(orow_v.at[0], pooled_hbm.at[base + _SCQ - 2],
                          sem_o).wait()
    pltpu.make_async_copy(orow_v.at[1], pooled_hbm.at[base + _SCQ - 1],
                          sem_o).wait()


def _run_sc_gather(nbr, p, n_p, x):
    nbr_b = nbr.reshape(_NW, _SCQ, _K)
    ntf_b = jnp.transpose(nbr_b, (0, 2, 1))
    px = p[:, 0]
    py = p[:, 1]
    pz = p[:, 2]
    npx = n_p[:, 0]
    npy = n_p[:, 1]
    npz = n_p[:, 2]
    mesh = plsc.VectorSubcoreMesh(core_axis_name="c", subcore_axis_name="s")
    f32 = jnp.float32
    kern = pl.kernel(
        _sc_gather_body,
        out_type=[
            jax.ShapeDtypeStruct((_M, 512), f32),
            jax.ShapeDtypeStruct((_M,), f32),
            jax.ShapeDtypeStruct((_M,), f32),
            jax.ShapeDtypeStruct((_M,), f32),
            jax.ShapeDtypeStruct((_M,), f32),
        ],
        mesh=mesh,
        scratch_types=[
            pltpu.VMEM((_SCQ, _K), jnp.int32),
            pltpu.VMEM((_K, _SCQ), jnp.int32),
            pltpu.VMEM((_K, _SCQ), f32),
            pltpu.VMEM((_K, _SCQ), f32),
            pltpu.VMEM((_K, _SCQ), f32),
            pltpu.VMEM((_SCQ,), f32),
            pltpu.VMEM((_SCQ,), f32),
            pltpu.VMEM((_SCQ,), f32),
            pltpu.VMEM((2, _K, 512), f32),
            pltpu.VMEM((2, 512), f32),
            pltpu.VMEM((_SCQ,), f32),
            pltpu.VMEM((_SCQ,), f32),
            pltpu.VMEM((_SCQ,), f32),
            pltpu.VMEM((_SCQ,), f32),
            pltpu.SemaphoreType.DMA,
            pltpu.SemaphoreType.DMA,
            pltpu.SemaphoreType.DMA,
        ],
    )
    return kern(nbr_b, ntf_b, px, py, pz, npx, npy, npz, x)


def _mlp_body(pooled_ref, mpx_ref, mpy_ref, mpz_ref, mx2_ref,
              w3_ref, wx_ref, b_ref, gamma_ref, beta_ref, out_ref):
    mx2 = mx2_ref[...]
    denom = jnp.sqrt(mx2) + jnp.float32(1e-8)
    f3 = jnp.concatenate(
        [mpx_ref[...] / denom, mpy_ref[...] / denom, mpz_ref[...] / denom],
        axis=1)
    h3 = jax.lax.dot_general(f3, w3_ref[...], (((1,), (0,)), ((), ())),
                             preferred_element_type=jnp.float32)
    hx = jax.lax.dot_general(pooled_ref[...], wx_ref[...],
                             (((1,), (0,)), ((), ())),
                             preferred_element_type=jnp.float32)
    h = h3 + hx + b_ref[...]
    mean = jnp.mean(h, axis=0, keepdims=True)
    c = h - mean
    var = jnp.mean(c * c, axis=0, keepdims=True)
    hn = c / jnp.sqrt(var + 1e-5) * gamma_ref[...] + beta_ref[...]
    out_ref[...] = jnp.maximum(hn, 0.0)


def _run_mlp(pooled, mpx, mpy, mpz, mx2, W, b, gamma, beta):
    w3 = W[:3]
    wx = W[3:]
    out = pl.pallas_call(
        _mlp_body,
        out_shape=jax.ShapeDtypeStruct((_M, 512), jnp.float32),
    )(pooled, mpx.reshape(_M, 1), mpy.reshape(_M, 1), mpz.reshape(_M, 1),
      mx2.reshape(_M, 1), w3, wx, b.reshape(1, 512),
      gamma.reshape(1, 512), beta.reshape(1, 512))
    return out


def kernel(p, x, o, W, b, gamma, beta):
    idx, n_p = _run_fps(p)
    nbr = _run_knn(n_p, p)
    pooled, mpx, mpy, mpz, mx2 = _run_sc_gather(nbr, p, n_p, x)
    out = _run_mlp(pooled, mpx, mpy, mpz, mx2, W, b, gamma, beta)
    n_o = o // 4
    return (n_p, out, n_o)
